# baseline (speedup 1.0000x reference)
import functools

import jax
import jax.numpy as jnp
from jax import lax
from jax.experimental import pallas as pl
from jax.experimental.pallas import tpu as pltpu
from jax.experimental.pallas import tpu_sc as plsc

B = 32
A = 8732
C = 21
O = 8
THRESH = 0.5
NEGPOS = 3
VAR0 = 0.1
VAR1 = 0.2
LN2 = 0.6931471805599453

NC, NS, L = 2, 16, 16
CHUNK = 128
NCHUNK = 68
EPI = A - NCHUNK * CHUNK
AP = 8736
NV = AP // L
UNR = 6


def _shuf(x, idx):
    dnums = lax.GatherDimensionNumbers(
        offset_dims=(), collapsed_slice_dims=(0,), start_index_map=(0,))
    return lax.gather(x, idx[:, None], dnums, (1,),
                      mode=lax.GatherScatterMode.PROMISE_IN_BOUNDS)


def _allmax(x, iota):
    for sh in (8, 4, 2, 1):
        x = jnp.maximum(x, _shuf(x, iota ^ sh))
    return x


def _allmin(x, iota):
    for sh in (8, 4, 2, 1):
        x = jnp.minimum(x, _shuf(x, iota ^ sh))
    return x


def _allsum(x, iota):
    for sh in (8, 4, 2, 1):
        x = x + _shuf(x, iota ^ sh)
    return x


def _prefix_excl(x, iota):
    y = x
    for sh in (1, 2, 4, 8):
        y = y + jnp.where(iota >= sh, _shuf(y, (iota - sh) & (L - 1)), 0)
    return y - x


def _vlog(x):
    bits = plsc.bitcast(x, jnp.int32)
    e = lax.shift_right_logical(bits, 23) - 127
    m = plsc.bitcast((bits & 0x7FFFFF) | 0x3F800000, jnp.float32)
    z = (m - 1.0) / (m + 1.0)
    z2 = z * z
    lm = 2.0 * z * (1.0 + z2 * (1.0 / 3.0 + z2 * (0.2 + z2 * (1.0 / 7.0))))
    return e.astype(jnp.float32) * LN2 + lm


def _sc_body(loc_hbm, conf_hbm, anch_hbm, cepi_hbm, lepi_hbm, ttab_hbm,
             l1_hbm, out_hbm,
             ttab_v, l1_v, bto_v, bti_v, lc_v,
             anch0, anch1, conf0, conf1, loc0, loc1, conf_e, loc_e,
             res_v, sem0, sem1):
    b = lax.axis_index("s") * NC + lax.axis_index("c")
    iota = lax.iota(jnp.int32, L)
    zi = jnp.zeros((L,), jnp.int32)

    pltpu.sync_copy(ttab_hbm.at[b], ttab_v)
    pltpu.sync_copy(l1_hbm.at[b], l1_v)

    r0, r1, r2, r3 = ttab_v[0, :], ttab_v[1, :], ttab_v[2, :], ttab_v[3, :]
    tx0 = [r0[j] for j in range(O)]
    ty0 = [r1[j] for j in range(O)]
    tx1 = [r2[j] for j in range(O)]
    ty1 = [r3[j] for j in range(O)]
    t_area = [(tx1[j] - tx0[j]) * (ty1[j] - ty0[j]) for j in range(O)]
    lr = l1_v[0, :]
    labs = [lr[j] for j in range(O)]

    def start_anch(c, dst, sem):
        pltpu.async_copy(anch_hbm.at[pl.ds(c * CHUNK, CHUNK)], dst, sem)

    def wait_anch(dst, sem):
        pltpu.make_async_copy(anch_hbm.at[pl.ds(0, CHUNK)], dst, sem).wait()

    def _group_a(base, rows, av, carry):
        bv = list(carry[:O])
        bi = list(carry[O:])
        acx = plsc.load_gather(av, [rows, zi])
        acy = plsc.load_gather(av, [rows, zi + 1])
        aw = plsc.load_gather(av, [rows, zi + 2])
        ah = plsc.load_gather(av, [rows, zi + 3])
        ax0 = acx - aw * 0.5
        ay0 = acy - ah * 0.5
        ax1 = acx + aw * 0.5
        ay1 = acy + ah * 0.5
        area_a = aw * ah
        idxv = base + iota
        validv = idxv < A
        btog = jnp.full((L,), -1.0, jnp.float32)
        btig = zi
        for j in range(O):
            iw = jnp.maximum(jnp.minimum(tx1[j], ax1) - jnp.maximum(tx0[j], ax0), 0.0)
            ih = jnp.maximum(jnp.minimum(ty1[j], ay1) - jnp.maximum(ty0[j], ay0), 0.0)
            inter = iw * ih
            iou = inter / (t_area[j] + area_a - inter)
            iou = jnp.where(validv, iou, -1.0)
            gt = iou > btog
            btog = jnp.where(gt, iou, btog)
            btig = jnp.where(gt, j, btig)
            gt2 = iou > bv[j]
            bv[j] = jnp.where(gt2, iou, bv[j])
            bi[j] = jnp.where(gt2, idxv, bi[j])
        bto_v[pl.ds(base, L)] = btog
        bti_v[pl.ds(base, L)] = btig
        return tuple(bv) + tuple(bi)

    def _chunk_a(c, av, carry):
        for g in range(CHUNK // L):
            carry = _group_a(c * CHUNK + g * L, g * L + iota, av, carry)
        return carry

    start_anch(0, anch0, sem0)
    init = tuple(jnp.full((L,), -1.0, jnp.float32) for _ in range(O)) + \
        tuple(zi for _ in range(O))

    def pass_a(i, carry):
        c0 = 2 * i
        start_anch(c0 + 1, anch1, sem1)
        wait_anch(anch0, sem0)
        carry = _chunk_a(c0, anch0, carry)

        @pl.when(c0 + 2 < NCHUNK)
        def _():
            start_anch(c0 + 2, anch0, sem0)

        wait_anch(anch1, sem1)
        carry = _chunk_a(c0 + 1, anch1, carry)
        return carry

    with jax.named_scope("sc_pass_a"):
        carry = lax.fori_loop(0, NCHUNK // 2, pass_a, init)
    pltpu.sync_copy(anch_hbm.at[pl.ds(NCHUNK * CHUNK, 32)],
                    anch0.at[pl.ds(0, 32)])
    for g in range(2):
        carry = _group_a(NCHUNK * CHUNK + g * L, g * L + iota, anch0, carry)

    bai = []
    for j in range(O):
        mxv = _allmax(carry[j], iota)
        cand = jnp.where(carry[j] == mxv, carry[O + j], AP)
        bai.append(_allmin(cand, iota)[0])

    def start_slot(c, cv, lv, av, sem):
        pltpu.async_copy(conf_hbm.at[b, pl.ds(c * CHUNK, CHUNK)], cv, sem)
        pltpu.async_copy(loc_hbm.at[b, pl.ds(c * CHUNK, CHUNK)], lv, sem)
        pltpu.async_copy(anch_hbm.at[pl.ds(c * CHUNK, CHUNK)], av, sem)

    def wait_slot(cv, lv, av, sem):
        pltpu.make_async_copy(conf_hbm.at[b, pl.ds(0, CHUNK)], cv, sem).wait()
        pltpu.make_async_copy(loc_hbm.at[b, pl.ds(0, CHUNK)], lv, sem).wait()
        pltpu.make_async_copy(anch_hbm.at[pl.ds(0, CHUNK)], av, sem).wait()

    def _group_b(base, rows, valid, cv, lv, av, npacc, llacc):
        btog = bto_v[pl.ds(base, L)]
        btig = bti_v[pl.ds(base, L)]
        idxv = base + iota
        for j in range(O):
            hit = idxv == bai[j]
            btog = jnp.where(hit, 2.0, btog)
            btig = jnp.where(hit, j, btig)
        pos = btog >= THRESH
        posf = jnp.where(pos, 1.0, 0.0)
        lab = jnp.full((L,), labs[O - 1], jnp.int32)
        for j in range(O - 2, -1, -1):
            lab = jnp.where(btig == j, labs[j], lab)
        conf_t = jnp.where(pos, lab, 0)
        xs = [plsc.load_gather(cv, [rows, zi + cc]) for cc in range(C)]
        m = xs[0]
        for cc in range(1, C):
            m = jnp.maximum(m, xs[cc])
        s = jnp.exp(xs[0] - m)
        for cc in range(1, C):
            s = s + jnp.exp(xs[cc] - m)
        lse = m + _vlog(s)
        xc = plsc.load_gather(cv, [rows, conf_t])
        ce = lse - xc
        lc = jnp.maximum(lse - xs[0], 0.0)
        lc = jnp.where(pos, 0.0, lc)
        if valid is not None:
            ce = jnp.where(valid, ce, 0.0)
            lc = jnp.where(valid, lc, 0.0)
            posf = jnp.where(valid, posf, 0.0)
        bti_v[pl.ds(base, L)] = plsc.bitcast(ce, jnp.int32)
        lc_v[pl.ds(base, L)] = lc
        bto_v[pl.ds(base, L)] = posf
        npacc = npacc + posf
        acx = plsc.load_gather(av, [rows, zi])
        acy = plsc.load_gather(av, [rows, zi + 1])
        aw = plsc.load_gather(av, [rows, zi + 2])
        ah = plsc.load_gather(av, [rows, zi + 3])
        m0 = plsc.load_gather(ttab_v, [zi, btig])
        m1 = plsc.load_gather(ttab_v, [zi + 1, btig])
        m2 = plsc.load_gather(ttab_v, [zi + 2, btig])
        m3 = plsc.load_gather(ttab_v, [zi + 3, btig])
        g0 = ((m0 + m2) * 0.5 - acx) / (aw * VAR0)
        g1 = ((m1 + m3) * 0.5 - acy) / (ah * VAR0)
        g2 = _vlog((m2 - m0) / aw + 1e-5) * (1.0 / VAR1)
        g3 = _vlog((m3 - m1) / ah + 1e-5) * (1.0 / VAR1)
        for k, gk in enumerate((g0, g1, g2, g3)):
            dk = plsc.load_gather(lv, [rows, zi + k]) - gk
            adk = jnp.abs(dk)
            sl1 = jnp.where(adk < 1.0, 0.5 * dk * dk, adk - 0.5)
            llacc = llacc + sl1 * posf
        return npacc, llacc

    def _chunk_b(c, cv, lv, av, npacc, llacc):
        for g in range(CHUNK // L):
            npacc, llacc = _group_b(c * CHUNK + g * L, g * L + iota, None,
                                    cv, lv, av, npacc, llacc)
        return npacc, llacc

    zero = jnp.zeros((L,), jnp.float32)
    start_slot(0, conf0, loc0, anch0, sem0)

    def pass_b(i, carry):
        npacc, llacc = carry
        c0 = 2 * i
        start_slot(c0 + 1, conf1, loc1, anch1, sem1)
        wait_slot(conf0, loc0, anch0, sem0)
        npacc, llacc = _chunk_b(c0, conf0, loc0, anch0, npacc, llacc)

        @pl.when(c0 + 2 < NCHUNK)
        def _():
            start_slot(c0 + 2, conf0, loc0, anch0, sem0)

        wait_slot(conf1, loc1, anch1, sem1)
        npacc, llacc = _chunk_b(c0 + 1, conf1, loc1, anch1, npacc, llacc)
        return npacc, llacc

    with jax.named_scope("sc_pass_b"):
        npacc, llacc = lax.fori_loop(0, NCHUNK // 2, pass_b, (zero, zero))
    pltpu.sync_copy(cepi_hbm.at[b], conf_e)
    pltpu.sync_copy(lepi_hbm.at[b], loc_e)
    pltpu.sync_copy(anch_hbm.at[pl.ds(NCHUNK * CHUNK, 32)],
                    anch0.at[pl.ds(0, 32)])
    for g in range(2):
        rows = g * L + iota
        valid = None
        if g == 1:
            valid = rows < EPI
            rows = jnp.minimum(rows, EPI - 1)
        npacc, llacc = _group_b(NCHUNK * CHUNK + g * L, rows, valid,
                                conf_e, loc_e, anch0, npacc, llacc)

    num_pos = _allsum(npacc, iota)[0]
    loss_l = _allsum(llacc, iota)[0]
    num_neg = jnp.minimum(NEGPOS * num_pos.astype(jnp.int32), A - 1)

    def count_ge(t):
        def cbody(i, acc):
            for k in range(UNR):
                bits = plsc.bitcast(lc_v[pl.ds(i * (UNR * L) + k * L, L)],
                                    jnp.int32)
                acc = acc + jnp.where(bits >= t, 1, 0)
            return acc
        acc = lax.fori_loop(0, NV // UNR, cbody, zi)
        return _allsum(acc, iota)[0]

    def bsearch(_, lohi):
        lo, hi = lohi
        mid = lo + (hi - lo) // 2
        ge = count_ge(mid) >= num_neg
        return jnp.where(ge, mid, lo), jnp.where(ge, hi, mid)

    with jax.named_scope("sc_rank"):
        vstar, _ = lax.fori_loop(0, 31, bsearch,
                                 (jnp.int32(0), jnp.int32(0x7F800000)))
        c_gt = count_ge(vstar + 1)
    need = num_neg - c_gt

    def final(i, carry):
        eqcnt, acc = carry
        for k in range(UNR):
            base = i * (UNR * L) + k * L
            bits = plsc.bitcast(lc_v[pl.ds(base, L)], jnp.int32)
            ce = plsc.bitcast(bti_v[pl.ds(base, L)], jnp.float32)
            posf = bto_v[pl.ds(base, L)]
            eq = bits == vstar
            eqi = jnp.where(eq, 1, 0)
            rank = eqcnt + _prefix_excl(eqi, iota)
            sel = (posf > 0.0) | (bits > vstar) | (eq & (rank < need))
            acc = acc + jnp.where(sel, ce, 0.0)
            eqcnt = eqcnt + _allsum(eqi, iota)[0]
        return eqcnt, acc

    with jax.named_scope("sc_final"):
        _, lcacc = lax.fori_loop(0, NV // UNR, final, (jnp.int32(0), zero))
    loss_c = _allsum(lcacc, iota)[0]

    res = jnp.where(iota == 0, loss_c,
                    jnp.where(iota == 1, loss_l,
                              jnp.where(iota == 2, num_pos, 0.0)))
    res_v[0, :] = res
    pltpu.sync_copy(res_v, out_hbm.at[b])


_sc_kernel = functools.partial(
    pl.kernel,
    out_type=jax.ShapeDtypeStruct((B, 1, L), jnp.float32),
    mesh=plsc.VectorSubcoreMesh(core_axis_name="c", subcore_axis_name="s",
                                num_cores=NC, num_subcores=NS),
    compiler_params=pltpu.CompilerParams(needs_layout_passes=False,
                                         use_tc_tiling_on_sc=False),
    scratch_types=[
        pltpu.VMEM((4, L), jnp.float32),
        pltpu.VMEM((1, L), jnp.int32),
        pltpu.VMEM((AP,), jnp.float32),
        pltpu.VMEM((AP,), jnp.int32),
        pltpu.VMEM((AP,), jnp.float32),
        pltpu.VMEM((CHUNK, 4), jnp.float32),
        pltpu.VMEM((CHUNK, 4), jnp.float32),
        pltpu.VMEM((CHUNK, C), jnp.float32),
        pltpu.VMEM((CHUNK, C), jnp.float32),
        pltpu.VMEM((CHUNK, 4), jnp.float32),
        pltpu.VMEM((CHUNK, 4), jnp.float32),
        pltpu.VMEM((EPI, C), jnp.float32),
        pltpu.VMEM((EPI, 4), jnp.float32),
        pltpu.VMEM((1, L), jnp.float32),
        pltpu.SemaphoreType.DMA,
        pltpu.SemaphoreType.DMA,
    ],
)(_sc_body)


def _tc_finish_body(p_ref, oc_ref, ol_ref):
    p = p_ref[...]
    lc = jnp.sum(p[:, 0:1])
    ll = jnp.sum(p[:, 1:2])
    n = jnp.sum(p[:, 2:3]) * C
    oc_ref[...] = jnp.full((8, 128), lc / n, jnp.float32)
    ol_ref[...] = jnp.full((8, 128), ll / n, jnp.float32)


_tc_finish = pl.pallas_call(
    _tc_finish_body,
    out_shape=(jax.ShapeDtypeStruct((8, 128), jnp.float32),
               jax.ShapeDtypeStruct((8, 128), jnp.float32)),
)


def kernel(loc_data, conf_data, anchors, targets):
    anch_p = jnp.pad(anchors, ((0, 4), (0, 0)))
    cepi = conf_data[:, NCHUNK * CHUNK:, :]
    lepi = loc_data[:, NCHUNK * CHUNK:, :]
    ttab = jnp.pad(jnp.swapaxes(targets[..., :4], 1, 2),
                   ((0, 0), (0, 0), (0, L - O)))
    l1 = jnp.pad(targets[..., 4].astype(jnp.int32) + 1,
                 ((0, 0), (0, L - O)))[:, None, :]
    partials = _sc_kernel(loc_data, conf_data, anch_p, cepi, lepi, ttab, l1)
    oc, ol = _tc_finish(partials.reshape(B, L))
    return oc[0, 0], ol[0, 0]


# speedup vs baseline: 2.0390x; 2.0390x over previous
import functools

import jax
import jax.numpy as jnp
from jax import lax
from jax.experimental import pallas as pl
from jax.experimental.pallas import tpu as pltpu
from jax.experimental.pallas import tpu_sc as plsc

B = 32
A = 8732
C = 21
O = 8
THRESH = 0.5
NEGPOS = 3
VAR0 = 0.1
VAR1 = 0.2
LN2 = 0.6931471805599453

NC, NS, L = 2, 16, 16
CHUNK = 64
NCHUNK = 136
EPI = A - NCHUNK * CHUNK
AP = 8736
NV = AP // L
UNR = 6


def _shuf(x, idx):
    dnums = lax.GatherDimensionNumbers(
        offset_dims=(), collapsed_slice_dims=(0,), start_index_map=(0,))
    return lax.gather(x, idx[:, None], dnums, (1,),
                      mode=lax.GatherScatterMode.PROMISE_IN_BOUNDS)


def _allmax(x, iota):
    for sh in (8, 4, 2, 1):
        x = jnp.maximum(x, _shuf(x, iota ^ sh))
    return x


def _allmin(x, iota):
    for sh in (8, 4, 2, 1):
        x = jnp.minimum(x, _shuf(x, iota ^ sh))
    return x


def _allsum(x, iota):
    for sh in (8, 4, 2, 1):
        x = x + _shuf(x, iota ^ sh)
    return x


def _prefix_excl(x, iota):
    y = x
    for sh in (1, 2, 4, 8):
        y = y + jnp.where(iota >= sh, _shuf(y, (iota - sh) & (L - 1)), 0)
    return y - x


def _vlog(x):
    bits = plsc.bitcast(x, jnp.int32)
    e = lax.shift_right_logical(bits, 23) - 127
    m = plsc.bitcast((bits & 0x7FFFFF) | 0x3F800000, jnp.float32)
    z = (m - 1.0) / (m + 1.0)
    z2 = z * z
    lm = 2.0 * z * (1.0 + z2 * (1.0 / 3.0 + z2 * (0.2 + z2 * (1.0 / 7.0))))
    return e.astype(jnp.float32) * LN2 + lm


def _sc_body(loc_hbm, conf_hbm, anch_hbm, cepi_hbm, lepi_hbm, ttab_hbm,
             l1_hbm, out_hbm,
             ttab_v, l1_v, bto_v, bti_v, lc_v,
             anch0, anch1, conf0, conf1, loc0, loc1, conf_e, loc_e,
             res_v, sem0, sem1):
    b = lax.axis_index("s") * NC + lax.axis_index("c")
    iota = lax.iota(jnp.int32, L)
    zi = jnp.zeros((L,), jnp.int32)

    pltpu.sync_copy(ttab_hbm.at[b], ttab_v)
    pltpu.sync_copy(l1_hbm.at[b], l1_v)

    r0, r1, r2, r3 = ttab_v[0, :], ttab_v[1, :], ttab_v[2, :], ttab_v[3, :]
    tx0 = [r0[j] for j in range(O)]
    ty0 = [r1[j] for j in range(O)]
    tx1 = [r2[j] for j in range(O)]
    ty1 = [r3[j] for j in range(O)]
    t_area = [(tx1[j] - tx0[j]) * (ty1[j] - ty0[j]) for j in range(O)]
    lr = l1_v[0, :]
    labs = [lr[j] for j in range(O)]

    def start_anch(c, dst, sem):
        pltpu.async_copy(anch_hbm.at[pl.ds(c * CHUNK, CHUNK)], dst, sem)

    def wait_anch(dst, sem):
        pltpu.make_async_copy(anch_hbm.at[pl.ds(0, CHUNK)], dst, sem).wait()

    def _group_a(base, rows, av, carry):
        bv = list(carry[:O])
        bi = list(carry[O:])
        acx = plsc.load_gather(av, [rows, zi])
        acy = plsc.load_gather(av, [rows, zi + 1])
        aw = plsc.load_gather(av, [rows, zi + 2])
        ah = plsc.load_gather(av, [rows, zi + 3])
        ax0 = acx - aw * 0.5
        ay0 = acy - ah * 0.5
        ax1 = acx + aw * 0.5
        ay1 = acy + ah * 0.5
        area_a = aw * ah
        idxv = base + iota
        validv = idxv < A
        btog = jnp.full((L,), -1.0, jnp.float32)
        btig = zi
        for j in range(O):
            iw = jnp.maximum(jnp.minimum(tx1[j], ax1) - jnp.maximum(tx0[j], ax0), 0.0)
            ih = jnp.maximum(jnp.minimum(ty1[j], ay1) - jnp.maximum(ty0[j], ay0), 0.0)
            inter = iw * ih
            iou = inter / (t_area[j] + area_a - inter)
            iou = jnp.where(validv, iou, -1.0)
            gt = iou > btog
            btog = jnp.where(gt, iou, btog)
            btig = jnp.where(gt, j, btig)
            gt2 = iou > bv[j]
            bv[j] = jnp.where(gt2, iou, bv[j])
            bi[j] = jnp.where(gt2, idxv, bi[j])
        bto_v[pl.ds(base, L)] = btog
        bti_v[pl.ds(base, L)] = btig
        return tuple(bv) + tuple(bi)

    def _chunk_a(c, av, carry):
        for g in range(CHUNK // L):
            carry = _group_a(c * CHUNK + g * L, g * L + iota, av, carry)
        return carry

    start_anch(0, anch0, sem0)
    init = tuple(jnp.full((L,), -1.0, jnp.float32) for _ in range(O)) + \
        tuple(zi for _ in range(O))

    def pass_a(i, carry):
        c0 = 2 * i
        start_anch(c0 + 1, anch1, sem1)
        wait_anch(anch0, sem0)
        carry = _chunk_a(c0, anch0, carry)

        @pl.when(c0 + 2 < NCHUNK)
        def _():
            start_anch(c0 + 2, anch0, sem0)

        wait_anch(anch1, sem1)
        carry = _chunk_a(c0 + 1, anch1, carry)
        return carry

    with jax.named_scope("sc_pass_a"):
        carry = lax.fori_loop(0, NCHUNK // 2, pass_a, init)
    pltpu.sync_copy(anch_hbm.at[pl.ds(NCHUNK * CHUNK, 32)],
                    anch0.at[pl.ds(0, 32)])
    for g in range(2):
        carry = _group_a(NCHUNK * CHUNK + g * L, g * L + iota, anch0, carry)

    bai = []
    for j in range(O):
        mxv = _allmax(carry[j], iota)
        cand = jnp.where(carry[j] == mxv, carry[O + j], AP)
        bai.append(_allmin(cand, iota)[0])

    def start_slot(c, cv, lv, av, sem):
        pltpu.async_copy(conf_hbm.at[b, pl.ds(c * CHUNK, CHUNK)], cv, sem)
        pltpu.async_copy(loc_hbm.at[b, pl.ds(c * CHUNK, CHUNK)], lv, sem)
        pltpu.async_copy(anch_hbm.at[pl.ds(c * CHUNK, CHUNK)], av, sem)

    def wait_slot(cv, lv, av, sem):
        pltpu.make_async_copy(conf_hbm.at[b, pl.ds(0, CHUNK)], cv, sem).wait()
        pltpu.make_async_copy(loc_hbm.at[b, pl.ds(0, CHUNK)], lv, sem).wait()
        pltpu.make_async_copy(anch_hbm.at[pl.ds(0, CHUNK)], av, sem).wait()

    def _group_b(base, rows, valid, cv, lv, av, npacc, llacc):
        btog = bto_v[pl.ds(base, L)]
        btig = bti_v[pl.ds(base, L)]
        idxv = base + iota
        for j in range(O):
            hit = idxv == bai[j]
            btog = jnp.where(hit, 2.0, btog)
            btig = jnp.where(hit, j, btig)
        pos = btog >= THRESH
        posf = jnp.where(pos, 1.0, 0.0)
        lab = jnp.full((L,), labs[O - 1], jnp.int32)
        for j in range(O - 2, -1, -1):
            lab = jnp.where(btig == j, labs[j], lab)
        conf_t = jnp.where(pos, lab, 0)
        xs = [plsc.load_gather(cv, [rows, zi + cc]) for cc in range(C)]
        m = xs[0]
        for cc in range(1, C):
            m = jnp.maximum(m, xs[cc])
        s = jnp.exp(xs[0] - m)
        for cc in range(1, C):
            s = s + jnp.exp(xs[cc] - m)
        lse = m + _vlog(s)
        xc = plsc.load_gather(cv, [rows, conf_t])
        ce = lse - xc
        lc = jnp.maximum(lse - xs[0], 0.0)
        lc = jnp.where(pos, 0.0, lc)
        if valid is not None:
            ce = jnp.where(valid, ce, 0.0)
            lc = jnp.where(valid, lc, 0.0)
            posf = jnp.where(valid, posf, 0.0)
        bti_v[pl.ds(base, L)] = plsc.bitcast(ce, jnp.int32)
        lc_v[pl.ds(base, L)] = lc
        bto_v[pl.ds(base, L)] = posf
        npacc = npacc + posf
        acx = plsc.load_gather(av, [rows, zi])
        acy = plsc.load_gather(av, [rows, zi + 1])
        aw = plsc.load_gather(av, [rows, zi + 2])
        ah = plsc.load_gather(av, [rows, zi + 3])
        m0 = plsc.load_gather(ttab_v, [zi, btig])
        m1 = plsc.load_gather(ttab_v, [zi + 1, btig])
        m2 = plsc.load_gather(ttab_v, [zi + 2, btig])
        m3 = plsc.load_gather(ttab_v, [zi + 3, btig])
        g0 = ((m0 + m2) * 0.5 - acx) / (aw * VAR0)
        g1 = ((m1 + m3) * 0.5 - acy) / (ah * VAR0)
        g2 = _vlog((m2 - m0) / aw + 1e-5) * (1.0 / VAR1)
        g3 = _vlog((m3 - m1) / ah + 1e-5) * (1.0 / VAR1)
        for k, gk in enumerate((g0, g1, g2, g3)):
            dk = plsc.load_gather(lv, [rows, zi + k]) - gk
            adk = jnp.abs(dk)
            sl1 = jnp.where(adk < 1.0, 0.5 * dk * dk, adk - 0.5)
            llacc = llacc + sl1 * posf
        return npacc, llacc

    def _chunk_b(c, cv, lv, av, npacc, llacc):
        for g in range(CHUNK // L):
            npacc, llacc = _group_b(c * CHUNK + g * L, g * L + iota, None,
                                    cv, lv, av, npacc, llacc)
        return npacc, llacc

    zero = jnp.zeros((L,), jnp.float32)
    start_slot(0, conf0, loc0, anch0, sem0)

    def pass_b(i, carry):
        npacc, llacc = carry
        c0 = 2 * i
        start_slot(c0 + 1, conf1, loc1, anch1, sem1)
        wait_slot(conf0, loc0, anch0, sem0)
        npacc, llacc = _chunk_b(c0, conf0, loc0, anch0, npacc, llacc)

        @pl.when(c0 + 2 < NCHUNK)
        def _():
            start_slot(c0 + 2, conf0, loc0, anch0, sem0)

        wait_slot(conf1, loc1, anch1, sem1)
        npacc, llacc = _chunk_b(c0 + 1, conf1, loc1, anch1, npacc, llacc)
        return npacc, llacc

    with jax.named_scope("sc_pass_b"):
        npacc, llacc = lax.fori_loop(0, NCHUNK // 2, pass_b, (zero, zero))
    pltpu.sync_copy(cepi_hbm.at[b], conf_e)
    pltpu.sync_copy(lepi_hbm.at[b], loc_e)
    pltpu.sync_copy(anch_hbm.at[pl.ds(NCHUNK * CHUNK, 32)],
                    anch0.at[pl.ds(0, 32)])
    for g in range(2):
        rows = g * L + iota
        valid = None
        if g == 1:
            valid = rows < EPI
            rows = jnp.minimum(rows, EPI - 1)
        npacc, llacc = _group_b(NCHUNK * CHUNK + g * L, rows, valid,
                                conf_e, loc_e, anch0, npacc, llacc)

    num_pos = _allsum(npacc, iota)[0]
    loss_l = _allsum(llacc, iota)[0]
    num_neg = jnp.minimum(NEGPOS * num_pos.astype(jnp.int32), A - 1)

    def count_ge(t):
        def cbody(i, acc):
            for k in range(UNR):
                bits = plsc.bitcast(lc_v[pl.ds(i * (UNR * L) + k * L, L)],
                                    jnp.int32)
                acc = acc + jnp.where(bits >= t, 1, 0)
            return acc
        acc = lax.fori_loop(0, NV // UNR, cbody, zi)
        return _allsum(acc, iota)[0]

    def bsearch(_, lohi):
        lo, hi = lohi
        mid = lo + (hi - lo) // 2
        ge = count_ge(mid) >= num_neg
        return jnp.where(ge, mid, lo), jnp.where(ge, hi, mid)

    with jax.named_scope("sc_rank"):
        vstar, _ = lax.fori_loop(0, 31, bsearch,
                                 (jnp.int32(0), jnp.int32(0x7F800000)))
        c_gt = count_ge(vstar + 1)
    need = num_neg - c_gt

    def final(i, carry):
        eqcnt, acc = carry
        for k in range(UNR):
            base = i * (UNR * L) + k * L
            bits = plsc.bitcast(lc_v[pl.ds(base, L)], jnp.int32)
            ce = plsc.bitcast(bti_v[pl.ds(base, L)], jnp.float32)
            posf = bto_v[pl.ds(base, L)]
            eq = bits == vstar
            eqi = jnp.where(eq, 1, 0)
            rank = eqcnt + _prefix_excl(eqi, iota)
            sel = (posf > 0.0) | (bits > vstar) | (eq & (rank < need))
            acc = acc + jnp.where(sel, ce, 0.0)
            eqcnt = eqcnt + _allsum(eqi, iota)[0]
        return eqcnt, acc

    with jax.named_scope("sc_final"):
        _, lcacc = lax.fori_loop(0, NV // UNR, final, (jnp.int32(0), zero))
    loss_c = _allsum(lcacc, iota)[0]

    res = jnp.where(iota == 0, loss_c,
                    jnp.where(iota == 1, loss_l,
                              jnp.where(iota == 2, num_pos, 0.0)))
    res_v[0, :] = res
    pltpu.sync_copy(res_v, out_hbm.at[b])


_sc_kernel = functools.partial(
    pl.kernel,
    out_type=jax.ShapeDtypeStruct((B, 1, L), jnp.float32),
    mesh=plsc.VectorSubcoreMesh(core_axis_name="c", subcore_axis_name="s",
                                num_cores=NC, num_subcores=NS),
    compiler_params=pltpu.CompilerParams(needs_layout_passes=False),
    scratch_types=[
        pltpu.VMEM((4, L), jnp.float32),
        pltpu.VMEM((1, L), jnp.int32),
        pltpu.VMEM((AP,), jnp.float32),
        pltpu.VMEM((AP,), jnp.int32),
        pltpu.VMEM((AP,), jnp.float32),
        pltpu.VMEM((CHUNK, 4), jnp.float32),
        pltpu.VMEM((CHUNK, 4), jnp.float32),
        pltpu.VMEM((CHUNK, C), jnp.float32),
        pltpu.VMEM((CHUNK, C), jnp.float32),
        pltpu.VMEM((CHUNK, 4), jnp.float32),
        pltpu.VMEM((CHUNK, 4), jnp.float32),
        pltpu.VMEM((EPI, C), jnp.float32),
        pltpu.VMEM((EPI, 4), jnp.float32),
        pltpu.VMEM((1, L), jnp.float32),
        pltpu.SemaphoreType.DMA,
        pltpu.SemaphoreType.DMA,
    ],
)(_sc_body)


def _tc_finish_body(p_ref, oc_ref, ol_ref):
    p = p_ref[...]
    lc = jnp.sum(p[:, 0:1])
    ll = jnp.sum(p[:, 1:2])
    n = jnp.sum(p[:, 2:3]) * C
    oc_ref[...] = jnp.full((8, 128), lc / n, jnp.float32)
    ol_ref[...] = jnp.full((8, 128), ll / n, jnp.float32)


_tc_finish = pl.pallas_call(
    _tc_finish_body,
    out_shape=(jax.ShapeDtypeStruct((8, 128), jnp.float32),
               jax.ShapeDtypeStruct((8, 128), jnp.float32)),
)


def kernel(loc_data, conf_data, anchors, targets):
    anch_p = jnp.pad(anchors, ((0, 4), (0, 0)))
    cepi = conf_data[:, NCHUNK * CHUNK:, :]
    lepi = loc_data[:, NCHUNK * CHUNK:, :]
    ttab = jnp.pad(jnp.swapaxes(targets[..., :4], 1, 2),
                   ((0, 0), (0, 0), (0, L - O)))
    l1 = jnp.pad(targets[..., 4].astype(jnp.int32) + 1,
                 ((0, 0), (0, L - O)))[:, None, :]
    partials = _sc_kernel(loc_data, conf_data, anch_p, cepi, lepi, ttab, l1)
    oc, ol = _tc_finish(partials.reshape(B, L))
    return oc[0, 0], ol[0, 0]


# speedup vs baseline: 4.2507x; 2.0847x over previous
import functools

import jax
import jax.numpy as jnp
from jax import lax
from jax.experimental import pallas as pl
from jax.experimental.pallas import tpu as pltpu
from jax.experimental.pallas import tpu_sc as plsc

B = 32
A = 8732
C = 21
O = 8
THRESH = 0.5
NEGPOS = 3
VAR0 = 0.1
VAR1 = 0.2
LN2 = 0.6931471805599453

NC, NS, L = 2, 16, 16
CHUNK = 128
NCHUNK = 68
EPI = A - NCHUNK * CHUNK
AP = 8736
NV = AP // L
UNR = 6
NG = CHUNK // L


def _shuf(x, idx):
    dnums = lax.GatherDimensionNumbers(
        offset_dims=(), collapsed_slice_dims=(0,), start_index_map=(0,))
    return lax.gather(x, idx[:, None], dnums, (1,),
                      mode=lax.GatherScatterMode.PROMISE_IN_BOUNDS)


def _allmax(x, iota):
    for sh in (8, 4, 2, 1):
        x = jnp.maximum(x, _shuf(x, iota ^ sh))
    return x


def _allmin(x, iota):
    for sh in (8, 4, 2, 1):
        x = jnp.minimum(x, _shuf(x, iota ^ sh))
    return x


def _allsum(x, iota):
    for sh in (8, 4, 2, 1):
        x = x + _shuf(x, iota ^ sh)
    return x


def _prefix_excl(x, iota):
    y = x
    for sh in (1, 2, 4, 8):
        y = y + jnp.where(iota >= sh, _shuf(y, (iota - sh) & (L - 1)), 0)
    return y - x


def _tree(op, vs):
    vs = list(vs)
    while len(vs) > 1:
        nxt = [op(vs[i], vs[i + 1]) for i in range(0, len(vs) - 1, 2)]
        if len(vs) % 2:
            nxt.append(vs[-1])
        vs = nxt
    return vs[0]


def _vlog(x):
    bits = plsc.bitcast(x, jnp.int32)
    e = lax.shift_right_logical(bits, 23) - 127
    m = plsc.bitcast((bits & 0x7FFFFF) | 0x3F800000, jnp.float32)
    z = (m - 1.0) / (m + 1.0)
    z2 = z * z
    lm = 2.0 * z * (1.0 + z2 * (1.0 / 3.0 + z2 * (0.2 + z2 * (1.0 / 7.0))))
    return e.astype(jnp.float32) * LN2 + lm


def _sc_body(loc_hbm, conf_hbm, anch_hbm, cepi_hbm, lepi_hbm, aepi_hbm,
             ttab_hbm, l1_hbm, out_hbm,
             ttab_v, l1_v, bto_v, bti_v, lc_v,
             anch0, anch1, conf0, conf1, loc0, loc1, conf_e, loc_e, anch_e,
             res_v, sem0, sem1):
    b = lax.axis_index("s") * NC + lax.axis_index("c")
    iota = lax.iota(jnp.int32, L)
    zi = jnp.zeros((L,), jnp.int32)

    pltpu.sync_copy(ttab_hbm.at[b], ttab_v)
    pltpu.sync_copy(l1_hbm.at[b], l1_v)
    pltpu.sync_copy(aepi_hbm, anch_e)

    r0, r1, r2, r3 = ttab_v[0, :], ttab_v[1, :], ttab_v[2, :], ttab_v[3, :]
    tx0 = [r0[j] for j in range(O)]
    ty0 = [r1[j] for j in range(O)]
    tx1 = [r2[j] for j in range(O)]
    ty1 = [r3[j] for j in range(O)]
    t_area = [(tx1[j] - tx0[j]) * (ty1[j] - ty0[j]) for j in range(O)]
    lr = l1_v[0, :]
    labs = [lr[j] for j in range(O)]

    def start_anch(c, dst, sem):
        pltpu.async_copy(anch_hbm.at[:, pl.ds(c * CHUNK, CHUNK)], dst, sem)

    def wait_anch(dst, sem):
        pltpu.make_async_copy(anch_hbm.at[:, pl.ds(0, CHUNK)], dst,
                              sem).wait()

    def _group_a(base, lbase, av, carry):
        bv = list(carry[:O])
        bi = list(carry[O:])
        acx = av[0, pl.ds(lbase, L)]
        acy = av[1, pl.ds(lbase, L)]
        aw = av[2, pl.ds(lbase, L)]
        ah = av[3, pl.ds(lbase, L)]
        ax0 = acx - aw * 0.5
        ay0 = acy - ah * 0.5
        ax1 = acx + aw * 0.5
        ay1 = acy + ah * 0.5
        area_a = aw * ah
        idxv = base + iota
        validv = idxv < A
        btog = jnp.full((L,), -1.0, jnp.float32)
        btig = zi
        for j in range(O):
            iw = jnp.maximum(jnp.minimum(tx1[j], ax1) - jnp.maximum(tx0[j], ax0), 0.0)
            ih = jnp.maximum(jnp.minimum(ty1[j], ay1) - jnp.maximum(ty0[j], ay0), 0.0)
            inter = iw * ih
            iou = inter / (t_area[j] + area_a - inter)
            iou = jnp.where(validv, iou, -1.0)
            gt = iou > btog
            btog = jnp.where(gt, iou, btog)
            btig = jnp.where(gt, j, btig)
            gt2 = iou > bv[j]
            bv[j] = jnp.where(gt2, iou, bv[j])
            bi[j] = jnp.where(gt2, idxv, bi[j])
        bto_v[pl.ds(base, L)] = btog
        bti_v[pl.ds(base, L)] = btig
        return tuple(bv) + tuple(bi)

    def _chunk_a(c, av, carry):
        for g in range(NG):
            carry = _group_a(c * CHUNK + g * L, g * L, av, carry)
        return carry

    start_anch(0, anch0, sem0)
    init = tuple(jnp.full((L,), -1.0, jnp.float32) for _ in range(O)) + \
        tuple(zi for _ in range(O))

    def pass_a(i, carry):
        c0 = 2 * i
        start_anch(c0 + 1, anch1, sem1)
        wait_anch(anch0, sem0)
        carry = _chunk_a(c0, anch0, carry)

        @pl.when(c0 + 2 < NCHUNK)
        def _():
            start_anch(c0 + 2, anch0, sem0)

        wait_anch(anch1, sem1)
        carry = _chunk_a(c0 + 1, anch1, carry)
        return carry

    with jax.named_scope("sc_pass_a"):
        carry = lax.fori_loop(0, NCHUNK // 2, pass_a, init)
    for g in range(2):
        carry = _group_a(NCHUNK * CHUNK + g * L, g * L, anch_e, carry)

    bai = []
    for j in range(O):
        mxv = _allmax(carry[j], iota)
        cand = jnp.where(carry[j] == mxv, carry[O + j], AP)
        bai.append(_allmin(cand, iota)[0])

    def start_slot(c, cv, lv, av, sem):
        pltpu.async_copy(conf_hbm.at[b, :, pl.ds(c * CHUNK, CHUNK)], cv, sem)
        pltpu.async_copy(loc_hbm.at[b, :, pl.ds(c * CHUNK, CHUNK)], lv, sem)
        pltpu.async_copy(anch_hbm.at[:, pl.ds(c * CHUNK, CHUNK)], av, sem)

    def wait_slot(cv, lv, av, sem):
        pltpu.make_async_copy(conf_hbm.at[b, :, pl.ds(0, CHUNK)],
                              cv, sem).wait()
        pltpu.make_async_copy(loc_hbm.at[b, :, pl.ds(0, CHUNK)],
                              lv, sem).wait()
        pltpu.make_async_copy(anch_hbm.at[:, pl.ds(0, CHUNK)],
                              av, sem).wait()

    def _group_b(base, lbase, valid, cv, lv, av, npacc, llacc):
        rows = lbase + iota
        btog = bto_v[pl.ds(base, L)]
        btig = bti_v[pl.ds(base, L)]
        idxv = base + iota
        for j in range(O):
            hit = idxv == bai[j]
            btog = jnp.where(hit, 2.0, btog)
            btig = jnp.where(hit, j, btig)
        pos = btog >= THRESH
        posf = jnp.where(pos, 1.0, 0.0)
        lab = jnp.full((L,), labs[O - 1], jnp.int32)
        for j in range(O - 2, -1, -1):
            lab = jnp.where(btig == j, labs[j], lab)
        conf_t = jnp.where(pos, lab, 0)
        xs = [cv[cc, pl.ds(lbase, L)] for cc in range(C)]
        m = _tree(jnp.maximum, xs)
        s = _tree(jnp.add, [jnp.exp(x - m) for x in xs])
        lse = m + _vlog(s)
        xc = plsc.load_gather(cv, [conf_t, rows])
        ce = lse - xc
        lc = jnp.maximum(lse - xs[0], 0.0)
        lc = jnp.where(pos, 0.0, lc)
        if valid is not None:
            ce = jnp.where(valid, ce, 0.0)
            lc = jnp.where(valid, lc, 0.0)
            posf = jnp.where(valid, posf, 0.0)
        bti_v[pl.ds(base, L)] = plsc.bitcast(ce, jnp.int32)
        lc_v[pl.ds(base, L)] = lc
        bto_v[pl.ds(base, L)] = posf
        npacc = npacc + posf
        acx = av[0, pl.ds(lbase, L)]
        acy = av[1, pl.ds(lbase, L)]
        aw = av[2, pl.ds(lbase, L)]
        ah = av[3, pl.ds(lbase, L)]
        m0 = plsc.load_gather(ttab_v, [zi, btig])
        m1 = plsc.load_gather(ttab_v, [zi + 1, btig])
        m2 = plsc.load_gather(ttab_v, [zi + 2, btig])
        m3 = plsc.load_gather(ttab_v, [zi + 3, btig])
        g0 = ((m0 + m2) * 0.5 - acx) / (aw * VAR0)
        g1 = ((m1 + m3) * 0.5 - acy) / (ah * VAR0)
        g2 = _vlog((m2 - m0) / aw + 1e-5) * (1.0 / VAR1)
        g3 = _vlog((m3 - m1) / ah + 1e-5) * (1.0 / VAR1)
        for k, gk in enumerate((g0, g1, g2, g3)):
            dk = lv[k, pl.ds(lbase, L)] - gk
            adk = jnp.abs(dk)
            sl1 = jnp.where(adk < 1.0, 0.5 * dk * dk, adk - 0.5)
            llacc = llacc + sl1 * posf
        return npacc, llacc

    def _chunk_b(c, cv, lv, av, npacc, llacc):
        for g in range(NG):
            npacc, llacc = _group_b(c * CHUNK + g * L, g * L, None,
                                    cv, lv, av, npacc, llacc)
        return npacc, llacc

    zero = jnp.zeros((L,), jnp.float32)
    start_slot(0, conf0, loc0, anch0, sem0)

    def pass_b(i, carry):
        npacc, llacc = carry
        c0 = 2 * i
        start_slot(c0 + 1, conf1, loc1, anch1, sem1)
        wait_slot(conf0, loc0, anch0, sem0)
        npacc, llacc = _chunk_b(c0, conf0, loc0, anch0, npacc, llacc)

        @pl.when(c0 + 2 < NCHUNK)
        def _():
            start_slot(c0 + 2, conf0, loc0, anch0, sem0)

        wait_slot(conf1, loc1, anch1, sem1)
        npacc, llacc = _chunk_b(c0 + 1, conf1, loc1, anch1, npacc, llacc)
        return npacc, llacc

    with jax.named_scope("sc_pass_b"):
        npacc, llacc = lax.fori_loop(0, NCHUNK // 2, pass_b, (zero, zero))
    pltpu.sync_copy(cepi_hbm.at[b], conf_e)
    pltpu.sync_copy(lepi_hbm.at[b], loc_e)
    for g in range(2):
        valid = None
        if g == 1:
            valid = g * L + iota < EPI
        npacc, llacc = _group_b(NCHUNK * CHUNK + g * L, g * L, valid,
                                conf_e, loc_e, anch_e, npacc, llacc)

    num_pos = _allsum(npacc, iota)[0]
    loss_l = _allsum(llacc, iota)[0]
    num_neg = jnp.minimum(NEGPOS * num_pos.astype(jnp.int32), A - 1)

    def count_ge(t):
        def cbody(i, acc):
            for k in range(UNR):
                bits = plsc.bitcast(lc_v[pl.ds(i * (UNR * L) + k * L, L)],
                                    jnp.int32)
                acc = acc + jnp.where(bits >= t, 1, 0)
            return acc
        acc = lax.fori_loop(0, NV // UNR, cbody, zi)
        return _allsum(acc, iota)[0]

    def bsearch(_, lohi):
        lo, hi = lohi
        mid = lo + (hi - lo) // 2
        ge = count_ge(mid) >= num_neg
        return jnp.where(ge, mid, lo), jnp.where(ge, hi, mid)

    with jax.named_scope("sc_rank"):
        vstar, _ = lax.fori_loop(0, 31, bsearch,
                                 (jnp.int32(0), jnp.int32(0x7F800000)))
        c_gt = count_ge(vstar + 1)
    need = num_neg - c_gt

    def final(i, carry):
        eqcnt, acc = carry
        for k in range(UNR):
            base = i * (UNR * L) + k * L
            bits = plsc.bitcast(lc_v[pl.ds(base, L)], jnp.int32)
            ce = plsc.bitcast(bti_v[pl.ds(base, L)], jnp.float32)
            posf = bto_v[pl.ds(base, L)]
            eq = bits == vstar
            eqi = jnp.where(eq, 1, 0)
            rank = eqcnt + _prefix_excl(eqi, iota)
            sel = (posf > 0.0) | (bits > vstar) | (eq & (rank < need))
            acc = acc + jnp.where(sel, ce, 0.0)
            eqcnt = eqcnt + _allsum(eqi, iota)[0]
        return eqcnt, acc

    with jax.named_scope("sc_final"):
        _, lcacc = lax.fori_loop(0, NV // UNR, final, (jnp.int32(0), zero))
    loss_c = _allsum(lcacc, iota)[0]

    res = jnp.where(iota == 0, loss_c,
                    jnp.where(iota == 1, loss_l,
                              jnp.where(iota == 2, num_pos, 0.0)))
    res_v[0, :] = res
    pltpu.sync_copy(res_v, out_hbm.at[b])


_sc_kernel = functools.partial(
    pl.kernel,
    out_type=jax.ShapeDtypeStruct((B, 1, L), jnp.float32),
    mesh=plsc.VectorSubcoreMesh(core_axis_name="c", subcore_axis_name="s",
                                num_cores=NC, num_subcores=NS),
    compiler_params=pltpu.CompilerParams(needs_layout_passes=False),
    scratch_types=[
        pltpu.VMEM((4, L), jnp.float32),
        pltpu.VMEM((1, L), jnp.int32),
        pltpu.VMEM((AP,), jnp.float32),
        pltpu.VMEM((AP,), jnp.int32),
        pltpu.VMEM((AP,), jnp.float32),
        pltpu.VMEM((4, CHUNK), jnp.float32),
        pltpu.VMEM((4, CHUNK), jnp.float32),
        pltpu.VMEM((C, CHUNK), jnp.float32),
        pltpu.VMEM((C, CHUNK), jnp.float32),
        pltpu.VMEM((4, CHUNK), jnp.float32),
        pltpu.VMEM((4, CHUNK), jnp.float32),
        pltpu.VMEM((C, 32), jnp.float32),
        pltpu.VMEM((4, 32), jnp.float32),
        pltpu.VMEM((4, 32), jnp.float32),
        pltpu.VMEM((1, L), jnp.float32),
        pltpu.SemaphoreType.DMA,
        pltpu.SemaphoreType.DMA,
    ],
)(_sc_body)


def _tc_finish_body(p_ref, oc_ref, ol_ref):
    p = p_ref[...]
    lc = jnp.sum(p[:, 0:1])
    ll = jnp.sum(p[:, 1:2])
    n = jnp.sum(p[:, 2:3]) * C
    oc_ref[...] = jnp.full((8, 128), lc / n, jnp.float32)
    ol_ref[...] = jnp.full((8, 128), ll / n, jnp.float32)


_tc_finish = pl.pallas_call(
    _tc_finish_body,
    out_shape=(jax.ShapeDtypeStruct((8, 128), jnp.float32),
               jax.ShapeDtypeStruct((8, 128), jnp.float32)),
)


def kernel(loc_data, conf_data, anchors, targets):
    loc_t = jnp.swapaxes(loc_data, 1, 2)
    conf_t = jnp.swapaxes(conf_data, 1, 2)
    anch_t = anchors.T
    cepi = jnp.pad(conf_t[:, :, NCHUNK * CHUNK:],
                   ((0, 0), (0, 0), (0, 32 - EPI)))
    lepi = jnp.pad(loc_t[:, :, NCHUNK * CHUNK:],
                   ((0, 0), (0, 0), (0, 32 - EPI)))
    aepi = jnp.pad(anch_t[:, NCHUNK * CHUNK:],
                   ((0, 0), (0, 32 - EPI)))
    ttab = jnp.pad(jnp.swapaxes(targets[..., :4], 1, 2),
                   ((0, 0), (0, 0), (0, L - O)))
    l1 = jnp.pad(targets[..., 4].astype(jnp.int32) + 1,
                 ((0, 0), (0, L - O)))[:, None, :]
    partials = _sc_kernel(loc_t, conf_t, anch_t, cepi, lepi, aepi, ttab, l1)
    oc, ol = _tc_finish(partials.reshape(B, L))
    return oc[0, 0], ol[0, 0]
